# baseline (speedup 1.0000x reference)
import functools

import jax
import jax.numpy as jnp
from jax import lax
from jax.experimental import pallas as pl
from jax.experimental.pallas import tpu as pltpu
from jax.experimental.pallas import tpu_sc as plsc

NUM_NODES = 1000000
OUT_DIMS = 32
BATCH = 16384
HIST = 200

SPLITS = ((0, 104), (104, 96))
NB = 4
CHUNK = NB * HIST
NC = 2
NS = 16
NW = NC * NS
BATCH_PER_W = BATCH // NW
CHUNKS_PER_W = BATCH_PER_W // NB


@functools.partial(
    pl.kernel,
    mesh=plsc.VectorSubcoreMesh(core_axis_name="c", subcore_axis_name="s"),
    compiler_params=pltpu.CompilerParams(use_tc_tiling_on_sc=False),
    out_type=jax.ShapeDtypeStruct((BATCH, HIST, OUT_DIMS), jnp.float32),
    scratch_types=[
        pltpu.VMEM((2, NB, HIST), jnp.int32),
        pltpu.VMEM((2, NB, HIST, OUT_DIMS), jnp.float32),
        pltpu.SemaphoreType.DMA,
        pltpu.SemaphoreType.DMA,
    ],
)
def _emb_lookup(idx_hbm, table_hbm, out_hbm, idx_v, rows_v, gsem, wsem):
    wid = lax.axis_index("s") * NC + lax.axis_index("c")
    b0_w = wid * BATCH_PER_W

    def fire(g, slot):
        pltpu.sync_copy(idx_hbm.at[pl.ds(b0_w + g * NB, NB)], idx_v.at[slot])
        for i in range(NB):
            for off, w in SPLITS:
                pltpu.async_copy(
                    table_hbm.at[idx_v.at[slot, i, pl.ds(off, w)]],
                    rows_v.at[slot, i, pl.ds(off, w)],
                    gsem,
                )

    def drain_gathers(slot):
        for i in range(NB):
            for off, w in SPLITS:
                pltpu.make_async_copy(
                    table_hbm.at[idx_v.at[slot, i, pl.ds(off, w)]],
                    rows_v.at[slot, i, pl.ds(off, w)],
                    gsem,
                ).wait()

    def out_slice(g):
        return out_hbm.at[pl.ds(b0_w + g * NB, NB)]

    fire(0, 0)

    def body(g, carry):
        s = g % 2
        ns = 1 - s

        @pl.when(g < CHUNKS_PER_W - 1)
        def _prefetch():
            @pl.when(g >= 1)
            def _reclaim():
                pltpu.make_async_copy(rows_v.at[ns], out_slice(g - 1), wsem).wait()

            fire(g + 1, ns)

        drain_gathers(s)
        pltpu.async_copy(rows_v.at[s], out_slice(g), wsem)
        return carry

    lax.fori_loop(0, CHUNKS_PER_W, body, 0)
    pltpu.make_async_copy(rows_v.at[0], out_slice(0), wsem).wait()
    pltpu.make_async_copy(rows_v.at[0], out_slice(0), wsem).wait()


def kernel(idx, table):
    return _emb_lookup(idx.astype(jnp.int32), table)
